# baseline (speedup 1.0000x reference)
import functools

import jax
import jax.numpy as jnp
from jax import lax
from jax.experimental import pallas as pl
from jax.experimental.pallas import tpu as pltpu
from jax.experimental.pallas import tpu_sc as plsc

N = 10000
E = 320000
D = 128
H = 128
C = 10
G = 128
L = 3
EPS = 1e-5

NP = 10240
PAD = N
EB = E // 128
EBP = 2560
GP = 144

f32 = jnp.float32
i32 = jnp.int32

_MESH = plsc.VectorSubcoreMesh(core_axis_name="c", subcore_axis_name="s")


def _bn(x):
    mean = jnp.mean(x, axis=0, keepdims=True)
    var = jnp.var(x, axis=0, keepdims=True)
    return (x - mean) / jnp.sqrt(var + EPS) + 0.0001


def _iota16():
    return lax.broadcasted_iota(i32, (16,), 0)


def _zero_acc(zeros_hbm, acc, s, rows_per_sub):
    pltpu.sync_copy(zeros_hbm, acc.at[pl.ds(s * rows_per_sub, rows_per_sub)])


def _writeback(acc, out, c, s, rows_per_sub):
    sl = pl.ds(s * rows_per_sub, rows_per_sub)
    pltpu.sync_copy(acc.at[sl], out.at[c, sl])


@functools.partial(
    pl.kernel,
    out_type=jax.ShapeDtypeStruct((2, NP, 16), f32),
    mesh=_MESH,
    compiler_params=pltpu.CompilerParams(use_tc_tiling_on_sc=False, needs_layout_passes=False),
    scratch_types=[
        pltpu.VMEM((4, 128), i32),
        pltpu.VMEM((128, 16), f32),
        pltpu.VMEM_SHARED((NP, 16), f32),
    ],
)
def _sc_degree(rowb, ones16, zeros16, out, ridx_v, wrow_v, acc):
    c = lax.axis_index("c")
    s = lax.axis_index("s")
    _zero_acc(zeros16.at[pl.ds(0, 640)], acc, s, 640)
    pltpu.sync_copy(ones16, wrow_v)
    plsc.subcore_barrier()

    w = c * 16 + s

    def step(t, _):
        blk = w * 80 + t * 4
        pltpu.sync_copy(rowb.at[pl.ds(blk, 4)], ridx_v)
        for j in range(4):
            pltpu.sync_copy(wrow_v, acc.at[ridx_v.at[j]], add=True)
        return 0

    lax.fori_loop(0, 20, step, 0)
    plsc.subcore_barrier()
    _writeback(acc, out, c, s, 640)


@functools.partial(
    pl.kernel,
    out_type=jax.ShapeDtypeStruct((2, NP, 64), f32),
    mesh=_MESH,
    compiler_params=pltpu.CompilerParams(use_tc_tiling_on_sc=False, needs_layout_passes=False),
    scratch_types=[
        pltpu.VMEM((4, 128), i32),
        pltpu.VMEM((4, 128), i32),
        pltpu.VMEM((512, 64), f32),
        pltpu.VMEM_SHARED((NP, 64), f32),
        pltpu.SemaphoreType.DMA,
        pltpu.SemaphoreType.DMA,
    ],
)
def _sc_conv(rowb, colb, y0_hbm, y1_hbm, zeros64, out,
             ridx_v, cidx_v, rows_v, acc, sem, sem2):
    c = lax.axis_index("c")
    s = lax.axis_index("s")
    _zero_acc(zeros64, acc, s, 640)
    plsc.subcore_barrier()

    def make_step(y_hbm):
        def step(t, _):
            blk = s * 160 + t * 4
            pltpu.sync_copy(rowb.at[pl.ds(blk, 4)], ridx_v)
            pltpu.sync_copy(colb.at[pl.ds(blk, 4)], cidx_v)
            gathers = []
            for j in range(4):
                dst = rows_v.at[pl.ds(j * 128, 128)]
                gathers.append(pltpu.async_copy(y_hbm.at[ridx_v.at[j]], dst, sem))
            scatters = []
            for j in range(4):
                gathers[j].wait()
                src = rows_v.at[pl.ds(j * 128, 128)]
                scatters.append(
                    pltpu.async_copy(src, acc.at[cidx_v.at[j]], sem2, add=True))
            for d in scatters:
                d.wait()
            return 0

        return step

    @pl.when(c == 0)
    def _():
        lax.fori_loop(0, 40, make_step(y0_hbm), 0)

    @pl.when(c == 1)
    def _():
        lax.fori_loop(0, 40, make_step(y1_hbm), 0)

    plsc.subcore_barrier()
    _writeback(acc, out, c, s, 640)


@functools.partial(
    pl.kernel,
    out_type=[
        jax.ShapeDtypeStruct((EBP * 128,), f32),
        jax.ShapeDtypeStruct((EBP * 128,), f32),
        jax.ShapeDtypeStruct((2, 2, NP, 16), f32),
    ],
    mesh=_MESH,
    compiler_params=pltpu.CompilerParams(use_tc_tiling_on_sc=False, needs_layout_passes=False),
    scratch_types=[
        pltpu.VMEM((2 * NP,), f32),
        pltpu.VMEM((4, 128), i32),
        pltpu.VMEM((4, 128), i32),
        pltpu.VMEM((512,), f32),
        pltpu.VMEM((512,), f32),
        pltpu.VMEM((128, 16), f32),
        pltpu.VMEM((128, 16), f32),
        pltpu.VMEM_SHARED((NP, 16), f32),
        pltpu.VMEM_SHARED((NP, 16), f32),
    ],
)
def _sc_att(drc_hbm, rowb, colb, zeros16, ewc_out, ewo_out, deg_out,
            drc_v, ridx_v, cidx_v, ewc_v, ewo_v, wrc_v, wro_v, accc, acco):
    c = lax.axis_index("c")
    s = lax.axis_index("s")
    _zero_acc(zeros16.at[pl.ds(0, 640)], accc, s, 640)
    _zero_acc(zeros16.at[pl.ds(0, 640)], acco, s, 640)
    pltpu.sync_copy(drc_hbm, drc_v)
    plsc.subcore_barrier()

    w = c * 16 + s

    def step(t, _):
        blk = w * 80 + t * 4
        pltpu.sync_copy(rowb.at[pl.ds(blk, 4)], ridx_v)
        pltpu.sync_copy(colb.at[pl.ds(blk, 4)], cidx_v)
        for j in range(4):
            for g in range(8):
                ir = ridx_v[j, pl.ds(16 * g, 16)]
                ic = cidx_v[j, pl.ds(16 * g, 16)]
                a = plsc.load_gather(drc_v, [ir * 2])
                b = plsc.load_gather(drc_v, [ic * 2 + 1])
                u = jnp.exp(-(a + b))
                ewc = 1.0 / (1.0 + u)
                ewo = u * ewc
                ewc_v[pl.ds(j * 128 + 16 * g, 16)] = ewc
                ewo_v[pl.ds(j * 128 + 16 * g, 16)] = ewo

            def fill(i, _):
                for d in range(8):
                    e = i * 8 + d
                    ei = jnp.full((16,), e, i32) + j * 128
                    wrc_v[e, pl.ds(0, 16)] = plsc.load_gather(ewc_v, [ei])
                    wro_v[e, pl.ds(0, 16)] = plsc.load_gather(ewo_v, [ei])
                return 0

            lax.fori_loop(0, 16, fill, 0)
            pltpu.sync_copy(wrc_v, accc.at[ridx_v.at[j]], add=True)
            pltpu.sync_copy(wro_v, acco.at[ridx_v.at[j]], add=True)
        pltpu.sync_copy(ewc_v, ewc_out.at[pl.ds(blk * 128, 512)])
        pltpu.sync_copy(ewo_v, ewo_out.at[pl.ds(blk * 128, 512)])
        return 0

    lax.fori_loop(0, 20, step, 0)
    plsc.subcore_barrier()
    sl = pl.ds(s * 640, 640)
    pltpu.sync_copy(accc.at[sl], deg_out.at[c, 0, sl])
    pltpu.sync_copy(acco.at[sl], deg_out.at[c, 1, sl])


@functools.partial(
    pl.kernel,
    out_type=jax.ShapeDtypeStruct((2, 2, NP, 64), f32),
    mesh=_MESH,
    compiler_params=pltpu.CompilerParams(use_tc_tiling_on_sc=False, needs_layout_passes=False),
    scratch_types=[
        pltpu.VMEM((4, 128), i32),
        pltpu.VMEM((4, 128), i32),
        pltpu.VMEM((512,), f32),
        pltpu.VMEM((512, 64), f32),
        pltpu.VMEM_SHARED((NP, 64), f32),
        pltpu.SemaphoreType.DMA,
        pltpu.SemaphoreType.DMA,
    ],
)
def _sc_conv_w(rowb, colb, ewc_hbm, ewo_hbm, yc0_hbm, yc1_hbm, yo0_hbm,
               yo1_hbm, zeros64, out, ridx_v, cidx_v, ew_v, rows_v, acc,
               sem, sem2):
    c = lax.axis_index("c")
    s = lax.axis_index("s")

    def run_pass(kind, half, ew_hbm, y_hbm):
        _zero_acc(zeros64, acc, s, 640)
        plsc.subcore_barrier()

        def step(t, _):
            blk = s * 160 + t * 4
            pltpu.sync_copy(rowb.at[pl.ds(blk, 4)], ridx_v)
            pltpu.sync_copy(colb.at[pl.ds(blk, 4)], cidx_v)
            pltpu.sync_copy(ew_hbm.at[pl.ds(blk * 128, 512)], ew_v)
            gathers = []
            for j in range(4):
                dst = rows_v.at[pl.ds(j * 128, 128)]
                gathers.append(pltpu.async_copy(y_hbm.at[ridx_v.at[j]], dst, sem))
            scatters = []
            for j in range(4):
                gathers[j].wait()

                def scale(i, _):
                    for d in range(16):
                        e = i * 16 + d
                        wv = plsc.load_gather(
                            ew_v, [jnp.full((16,), e, i32) + j * 128])
                        r = j * 128 + e
                        for l in range(4):
                            fs = pl.ds(16 * l, 16)
                            rows_v[r, fs] = rows_v[r, fs] * wv
                    return 0

                lax.fori_loop(0, 8, scale, 0)
                src = rows_v.at[pl.ds(j * 128, 128)]
                scatters.append(
                    pltpu.async_copy(src, acc.at[cidx_v.at[j]], sem2, add=True))
            for d in scatters:
                d.wait()
            return 0

        lax.fori_loop(0, 40, step, 0)
        plsc.subcore_barrier()
        sl = pl.ds(s * 640, 640)
        pltpu.sync_copy(acc.at[sl], out.at[kind, half, sl])
        plsc.subcore_barrier()

    @pl.when(c == 0)
    def _():
        run_pass(0, 0, ewc_hbm, yc0_hbm)
        run_pass(0, 1, ewc_hbm, yc1_hbm)

    @pl.when(c == 1)
    def _():
        run_pass(1, 0, ewo_hbm, yo0_hbm)
        run_pass(1, 1, ewo_hbm, yo1_hbm)


@functools.partial(
    pl.kernel,
    out_type=jax.ShapeDtypeStruct((2, GP, H), f32),
    mesh=_MESH,
    compiler_params=pltpu.CompilerParams(use_tc_tiling_on_sc=False, needs_layout_passes=False),
    scratch_types=[
        pltpu.VMEM((1, 128), i32),
        pltpu.VMEM((128, H), f32),
        pltpu.VMEM_SHARED((GP, H), f32),
    ],
)
def _sc_pool(xc_hbm, xo_hbm, batchb, zeros, out, bidx_v, data_v, acc):
    c = lax.axis_index("c")
    s = lax.axis_index("s")
    _zero_acc(zeros.at[pl.ds(0, 9)], acc, s, 9)
    plsc.subcore_barrier()

    def make_step(x_hbm):
        def step(t, _):
            row = s * 5 + t
            pltpu.sync_copy(batchb.at[pl.ds(row, 1)], bidx_v)
            pltpu.sync_copy(x_hbm.at[pl.ds(row * 128, 128)], data_v)
            pltpu.sync_copy(data_v, acc.at[bidx_v.at[0]], add=True)
            return 0

        return step

    @pl.when(c == 0)
    def _():
        lax.fori_loop(0, 5, make_step(xc_hbm), 0)

    @pl.when(c == 1)
    def _():
        lax.fori_loop(0, 5, make_step(xo_hbm), 0)

    plsc.subcore_barrier()
    _writeback(acc, out, c, s, 9)


def _dot(a, b):
    return jnp.dot(a, b, preferred_element_type=f32)


def _tc_call(body, out_shape):
    return pl.pallas_call(body, out_shape=out_shape)


def _write_y_halves(y_ref, y):
    zpad = jnp.zeros((NP - N, 64), f32)
    y_ref[0, pl.ds(0, N), :] = y[:, :64]
    y_ref[0, pl.ds(N, NP - N), :] = zpad
    y_ref[1, pl.ds(0, N), :] = y[:, 64:]
    y_ref[1, pl.ds(N, NP - N), :] = zpad


def _read_halves(ref, idx=None):
    if idx is None:
        lo = ref[0, pl.ds(0, N), :]
        hi = ref[1, pl.ds(0, N), :]
    else:
        lo = ref[idx, 0, pl.ds(0, N), :]
        hi = ref[idx, 1, pl.ds(0, N), :]
    return jnp.concatenate([lo, hi], axis=1)


def _k0_body(x_ref, degp_ref, wf_ref, bf_ref, w0_ref, y_ref, dinv_ref):
    deg = 1.0 + degp_ref[0, pl.ds(0, N), pl.ds(0, 1)] \
              + degp_ref[1, pl.ds(0, N), pl.ds(0, 1)]
    dinv = lax.rsqrt(deg)
    dinv_ref[...] = dinv
    x0 = jax.nn.relu(_dot(_bn(x_ref[...]), wf_ref[...]) + bf_ref[...])
    y1 = dinv * _dot(_bn(x0), w0_ref[...])
    _write_y_halves(y_ref, y1)


def _kmid_body(z_ref, y_ref, dinv_ref, b_ref, wn_ref, yn_ref):
    dinv = dinv_ref[...]
    x = jax.nn.relu(dinv * (_read_halves(z_ref) + _read_halves(y_ref))
                    + b_ref[...])
    yn = dinv * _dot(_bn(x), wn_ref[...])
    _write_y_halves(yn_ref, yn)


def _k3a_body(z_ref, y_ref, dinv_ref, b_ref, dur_ref, duc_ref, bd_ref,
              wna_ref, bna_ref, x3_ref, na_ref, drc_ref):
    dinv = dinv_ref[...]
    x3 = jax.nn.relu(dinv * (_read_halves(z_ref) + _read_halves(y_ref))
                     + b_ref[...])
    x3_ref[...] = x3
    na_ref[...] = jax.nn.softmax(_dot(x3, wna_ref[...]) + bna_ref[...], axis=-1)
    dr = _dot(x3, dur_ref[...]) + bd_ref[...]
    dc = _dot(x3, duc_ref[...])
    drc_ref[pl.ds(0, N), :] = jnp.concatenate([dr, dc], axis=1)
    drc_ref[pl.ds(N, NP - N), :] = jnp.zeros((NP - N, 2), f32)


def _k3b_body(x3_ref, na_ref, wc_ref, wo_ref, hc_ref, ho_ref):
    x3 = x3_ref[...]
    xc = na_ref[:, pl.ds(0, 1)] * x3
    xo = na_ref[:, pl.ds(1, 1)] * x3
    hc_ref[...] = _dot(_bn(xc), wc_ref[...])
    ho_ref[...] = _dot(_bn(xo), wo_ref[...])


def _k4a_body(degw_ref, dinvc_ref, dinvo_ref):
    degc = 1.0 + degw_ref[0, 0, pl.ds(0, N), pl.ds(0, 1)] \
               + degw_ref[1, 0, pl.ds(0, N), pl.ds(0, 1)]
    dego = 1.0 + degw_ref[0, 1, pl.ds(0, N), pl.ds(0, 1)] \
               + degw_ref[1, 1, pl.ds(0, N), pl.ds(0, 1)]
    dinvc_ref[...] = lax.rsqrt(degc)
    dinvo_ref[...] = lax.rsqrt(dego)


def _k4b_body(h_ref, dinv_ref, y2_ref):
    y = dinv_ref[...] * h_ref[...]
    _write_y_halves(y2_ref, y)


def _k5_body(zw_ref, y2_ref, dinv_ref, b_ref, xk_ref):
    zpad = jnp.zeros((NP - N, H), f32)
    xk = jax.nn.relu(dinv_ref[...] * (_read_halves(zw_ref) + _read_halves(y2_ref))
                     + b_ref[...])
    xk_ref[pl.ds(0, N), :] = xk
    xk_ref[pl.ds(N, NP - N), :] = zpad


def _readout(h, w1, b1, w2, b2):
    h = jax.nn.relu(_dot(_bn(h), w1) + b1)
    h = _dot(_bn(h), w2) + b2
    return jax.nn.log_softmax(h, axis=-1)


def _k6_body(pooled_ref, wc1_ref, bc1_ref, wc2_ref, bc2_ref,
             wo1_ref, bo1_ref, wo2_ref, bo2_ref,
             wco1_ref, bco1_ref, wco2_ref, bco2_ref,
             outc_ref, outo_ref, outco_ref):
    xc = pooled_ref[0, pl.ds(0, G), :]
    xo = pooled_ref[1, pl.ds(0, G), :]
    outc_ref[...] = _readout(xc, wc1_ref[...], bc1_ref[...],
                             wc2_ref[...], bc2_ref[...])
    outo_ref[...] = _readout(xo, wo1_ref[...], bo1_ref[...],
                             wo2_ref[...], bo2_ref[...])
    outco_ref[...] = _readout(xc + xo, wco1_ref[...], bco1_ref[...],
                              wco2_ref[...], bco2_ref[...])


def kernel(x, params, edge_index, batch):
    row = edge_index[0]
    col = edge_index[1]
    padE = jnp.full((EBP * 128 - E,), PAD, i32)
    rowb = jnp.concatenate([row, padE]).reshape(EBP, 128)
    colb = jnp.concatenate([col, padE]).reshape(EBP, 128)
    batchb = jnp.concatenate(
        [batch.astype(i32), jnp.full((NP - N,), G, i32)]).reshape(80, 128)

    zeros = jnp.zeros((640, H), f32)
    zeros64 = jnp.zeros((640, 64), f32)
    zeros16 = jnp.zeros((640, 16), f32)
    ones16 = jnp.ones((128, 16), f32)

    p = params
    row1 = lambda v: v.reshape(1, -1)

    degp = _sc_degree(rowb, ones16, zeros16)

    yhalf = jax.ShapeDtypeStruct((2, NP, 64), f32)

    y1, dinv = _tc_call(
        _k0_body,
        (yhalf, jax.ShapeDtypeStruct((N, 1), f32)),
    )(x, degp, p["W_feat"], row1(p["b_feat"]), p["W_convs"][0])

    z1 = _sc_conv(rowb, colb, y1[0], y1[1], zeros64)
    y2 = _tc_call(_kmid_body, yhalf)(
        z1, y1, dinv, row1(p["b_convs"][0]), p["W_convs"][1])
    z2 = _sc_conv(rowb, colb, y2[0], y2[1], zeros64)
    y3 = _tc_call(_kmid_body, yhalf)(
        z2, y2, dinv, row1(p["b_convs"][1]), p["W_convs"][2])
    z3 = _sc_conv(rowb, colb, y3[0], y3[1], zeros64)

    we = p["W_edge_att"]
    du = we[:, 0] - we[:, 1]
    dur = du[:H].reshape(H, 1)
    duc = du[H:].reshape(H, 1)
    bd = (p["b_edge_att"][0] - p["b_edge_att"][1]).reshape(1, 1)

    x3, na, drc = _tc_call(
        _k3a_body,
        (jax.ShapeDtypeStruct((N, H), f32),
         jax.ShapeDtypeStruct((N, 2), f32),
         jax.ShapeDtypeStruct((NP, 2), f32)),
    )(z3, y3, dinv, row1(p["b_convs"][2]), dur, duc, bd,
      p["W_node_att"], row1(p["b_node_att"]))

    ewc, ewo, degw = _sc_att(drc.reshape(2 * NP), rowb, colb, zeros16)

    hc, ho = _tc_call(
        _k3b_body,
        (jax.ShapeDtypeStruct((N, H), f32), jax.ShapeDtypeStruct((N, H), f32)),
    )(x3, na, p["W_ctx"], p["W_obj"])

    dinvc, dinvo = _tc_call(
        _k4a_body,
        (jax.ShapeDtypeStruct((N, 1), f32), jax.ShapeDtypeStruct((N, 1), f32)),
    )(degw)
    yc2 = _tc_call(_k4b_body, yhalf)(hc, dinvc)
    yo2 = _tc_call(_k4b_body, yhalf)(ho, dinvo)

    zw = _sc_conv_w(rowb, colb, ewc, ewo,
                    yc2[0], yc2[1], yo2[0], yo2[1], zeros64)

    npH = jax.ShapeDtypeStruct((NP, H), f32)
    xc = _tc_call(_k5_body, npH)(zw[0], yc2, dinvc, row1(p["b_ctx"]))
    xo = _tc_call(_k5_body, npH)(zw[1], yo2, dinvo, row1(p["b_obj"]))

    pooled = _sc_pool(xc, xo, batchb, zeros)

    outc, outo, outco = _tc_call(
        _k6_body,
        (jax.ShapeDtypeStruct((G, C), f32),
         jax.ShapeDtypeStruct((G, C), f32),
         jax.ShapeDtypeStruct((G, C), f32)),
    )(pooled,
      p["W_fc1_c"], row1(p["b_fc1_c"]), p["W_fc2_c"], row1(p["b_fc2_c"]),
      p["W_fc1_o"], row1(p["b_fc1_o"]), p["W_fc2_o"], row1(p["b_fc2_o"]),
      p["W_fc1_co"], row1(p["b_fc1_co"]), p["W_fc2_co"], row1(p["b_fc2_co"]))

    return outc, outo, outco


# speedup vs baseline: 1.1719x; 1.1719x over previous
import functools

import jax
import jax.numpy as jnp
from jax import lax
from jax.experimental import pallas as pl
from jax.experimental.pallas import tpu as pltpu
from jax.experimental.pallas import tpu_sc as plsc

N = 10000
E = 320000
D = 128
H = 128
C = 10
G = 128
L = 3
EPS = 1e-5

NP = 10240
PAD = N
EB = E // 128
EBP = 2560
GP = 144

f32 = jnp.float32
i32 = jnp.int32

_MESH = plsc.VectorSubcoreMesh(core_axis_name="c", subcore_axis_name="s")


def _bn(x):
    mean = jnp.mean(x, axis=0, keepdims=True)
    var = jnp.var(x, axis=0, keepdims=True)
    return (x - mean) / jnp.sqrt(var + EPS) + 0.0001


def _iota16():
    return lax.broadcasted_iota(i32, (16,), 0)


def _zero_acc(zeros_hbm, acc, s, rows_per_sub):
    pltpu.sync_copy(zeros_hbm, acc.at[pl.ds(s * rows_per_sub, rows_per_sub)])


def _writeback(acc, out, c, s, rows_per_sub):
    sl = pl.ds(s * rows_per_sub, rows_per_sub)
    pltpu.sync_copy(acc.at[sl], out.at[c, sl])


@functools.partial(
    pl.kernel,
    out_type=jax.ShapeDtypeStruct((2, NP, 16), f32),
    mesh=_MESH,
    compiler_params=pltpu.CompilerParams(use_tc_tiling_on_sc=False, needs_layout_passes=False),
    scratch_types=[
        pltpu.VMEM((4, 128), i32),
        pltpu.VMEM((128, 16), f32),
        pltpu.VMEM_SHARED((NP, 16), f32),
    ],
)
def _sc_degree(rowb, ones16, zeros16, out, ridx_v, wrow_v, acc):
    c = lax.axis_index("c")
    s = lax.axis_index("s")
    _zero_acc(zeros16.at[pl.ds(0, 640)], acc, s, 640)
    pltpu.sync_copy(ones16, wrow_v)
    plsc.subcore_barrier()

    w = c * 16 + s

    def step(t, _):
        blk = w * 80 + t * 4
        pltpu.sync_copy(rowb.at[pl.ds(blk, 4)], ridx_v)
        for j in range(4):
            pltpu.sync_copy(wrow_v, acc.at[ridx_v.at[j]], add=True)
        return 0

    lax.fori_loop(0, 20, step, 0)
    plsc.subcore_barrier()
    _writeback(acc, out, c, s, 640)


@functools.partial(
    pl.kernel,
    out_type=jax.ShapeDtypeStruct((2, NP, 64), f32),
    mesh=_MESH,
    compiler_params=pltpu.CompilerParams(use_tc_tiling_on_sc=False, needs_layout_passes=False),
    scratch_types=[
        pltpu.VMEM((4, 128), i32),
        pltpu.VMEM((4, 128), i32),
        pltpu.VMEM((512, 64), f32),
        pltpu.VMEM_SHARED((NP, 64), f32),
        pltpu.SemaphoreType.DMA,
        pltpu.SemaphoreType.DMA,
    ],
)
def _sc_conv(rowb, colb, y0_hbm, y1_hbm, zeros64, out,
             ridx_v, cidx_v, rows_v, acc, sem, sem2):
    c = lax.axis_index("c")
    s = lax.axis_index("s")
    _zero_acc(zeros64, acc, s, 640)
    plsc.subcore_barrier()

    def make_step(y_hbm):
        def step(t, _):
            blk = s * 160 + t * 4
            pltpu.sync_copy(rowb.at[pl.ds(blk, 4)], ridx_v)
            pltpu.sync_copy(colb.at[pl.ds(blk, 4)], cidx_v)
            gathers = []
            for j in range(4):
                dst = rows_v.at[pl.ds(j * 128, 128)]
                gathers.append(pltpu.async_copy(y_hbm.at[ridx_v.at[j]], dst, sem))
            scatters = []
            for j in range(4):
                gathers[j].wait()
                src = rows_v.at[pl.ds(j * 128, 128)]
                scatters.append(
                    pltpu.async_copy(src, acc.at[cidx_v.at[j]], sem2, add=True))
            for d in scatters:
                d.wait()
            return 0

        return step

    @pl.when(c == 0)
    def _():
        lax.fori_loop(0, 40, make_step(y0_hbm), 0)

    @pl.when(c == 1)
    def _():
        lax.fori_loop(0, 40, make_step(y1_hbm), 0)

    plsc.subcore_barrier()
    _writeback(acc, out, c, s, 640)


@functools.partial(
    pl.kernel,
    out_type=[
        jax.ShapeDtypeStruct((EBP * 128,), f32),
        jax.ShapeDtypeStruct((EBP * 128,), f32),
        jax.ShapeDtypeStruct((2, 2, NP, 16), f32),
    ],
    mesh=_MESH,
    compiler_params=pltpu.CompilerParams(use_tc_tiling_on_sc=False, needs_layout_passes=False),
    scratch_types=[
        pltpu.VMEM((2 * NP,), f32),
        pltpu.VMEM((4, 128), i32),
        pltpu.VMEM((4, 128), i32),
        pltpu.VMEM((512,), f32),
        pltpu.VMEM((512,), f32),
        pltpu.VMEM((128, 16), f32),
        pltpu.VMEM((128, 16), f32),
        pltpu.VMEM_SHARED((NP, 16), f32),
        pltpu.VMEM_SHARED((NP, 16), f32),
    ],
)
def _sc_att(drc_hbm, rowb, colb, zeros16, ewc_out, ewo_out, deg_out,
            drc_v, ridx_v, cidx_v, ewc_v, ewo_v, wrc_v, wro_v, accc, acco):
    c = lax.axis_index("c")
    s = lax.axis_index("s")
    _zero_acc(zeros16.at[pl.ds(0, 640)], accc, s, 640)
    _zero_acc(zeros16.at[pl.ds(0, 640)], acco, s, 640)
    pltpu.sync_copy(drc_hbm, drc_v)
    plsc.subcore_barrier()

    w = c * 16 + s

    def step(t, _):
        blk = w * 80 + t * 4
        pltpu.sync_copy(rowb.at[pl.ds(blk, 4)], ridx_v)
        pltpu.sync_copy(colb.at[pl.ds(blk, 4)], cidx_v)
        for j in range(4):
            for g in range(8):
                ir = ridx_v[j, pl.ds(16 * g, 16)]
                ic = cidx_v[j, pl.ds(16 * g, 16)]
                a = plsc.load_gather(drc_v, [ir * 2])
                b = plsc.load_gather(drc_v, [ic * 2 + 1])
                u = jnp.exp(-(a + b))
                ewc = 1.0 / (1.0 + u)
                ewo = u * ewc
                ewc_v[pl.ds(j * 128 + 16 * g, 16)] = ewc
                ewo_v[pl.ds(j * 128 + 16 * g, 16)] = ewo

            @functools.partial(plsc.parallel_loop, 0, 128, unroll=8)
            def _(e):
                ei = jnp.full((16,), e, i32) + j * 128
                wrc_v[e, pl.ds(0, 16)] = plsc.load_gather(ewc_v, [ei])
                wro_v[e, pl.ds(0, 16)] = plsc.load_gather(ewo_v, [ei])
            pltpu.sync_copy(wrc_v, accc.at[ridx_v.at[j]], add=True)
            pltpu.sync_copy(wro_v, acco.at[ridx_v.at[j]], add=True)
        pltpu.sync_copy(ewc_v, ewc_out.at[pl.ds(blk * 128, 512)])
        pltpu.sync_copy(ewo_v, ewo_out.at[pl.ds(blk * 128, 512)])
        return 0

    lax.fori_loop(0, 20, step, 0)
    plsc.subcore_barrier()
    sl = pl.ds(s * 640, 640)
    pltpu.sync_copy(accc.at[sl], deg_out.at[c, 0, sl])
    pltpu.sync_copy(acco.at[sl], deg_out.at[c, 1, sl])


@functools.partial(
    pl.kernel,
    out_type=jax.ShapeDtypeStruct((2, 2, NP, 64), f32),
    mesh=_MESH,
    compiler_params=pltpu.CompilerParams(use_tc_tiling_on_sc=False, needs_layout_passes=False),
    scratch_types=[
        pltpu.VMEM((4, 128), i32),
        pltpu.VMEM((4, 128), i32),
        pltpu.VMEM((512,), f32),
        pltpu.VMEM((512, 64), f32),
        pltpu.VMEM_SHARED((NP, 64), f32),
        pltpu.SemaphoreType.DMA,
        pltpu.SemaphoreType.DMA,
    ],
)
def _sc_conv_w(rowb, colb, ewc_hbm, ewo_hbm, yc0_hbm, yc1_hbm, yo0_hbm,
               yo1_hbm, zeros64, out, ridx_v, cidx_v, ew_v, rows_v, acc,
               sem, sem2):
    c = lax.axis_index("c")
    s = lax.axis_index("s")

    def run_pass(kind, half, ew_hbm, y_hbm):
        _zero_acc(zeros64, acc, s, 640)
        plsc.subcore_barrier()

        def step(t, _):
            blk = s * 160 + t * 4
            pltpu.sync_copy(rowb.at[pl.ds(blk, 4)], ridx_v)
            pltpu.sync_copy(colb.at[pl.ds(blk, 4)], cidx_v)
            pltpu.sync_copy(ew_hbm.at[pl.ds(blk * 128, 512)], ew_v)
            gathers = []
            for j in range(4):
                dst = rows_v.at[pl.ds(j * 128, 128)]
                gathers.append(pltpu.async_copy(y_hbm.at[ridx_v.at[j]], dst, sem))
            scatters = []
            for j in range(4):
                gathers[j].wait()

                @functools.partial(plsc.parallel_loop, 0, 128, unroll=8)
                def _(e):
                    wv = plsc.load_gather(
                        ew_v, [jnp.full((16,), e, i32) + j * 128])
                    r = j * 128 + e
                    for l in range(4):
                        fs = pl.ds(16 * l, 16)
                        rows_v[r, fs] = rows_v[r, fs] * wv
                src = rows_v.at[pl.ds(j * 128, 128)]
                scatters.append(
                    pltpu.async_copy(src, acc.at[cidx_v.at[j]], sem2, add=True))
            for d in scatters:
                d.wait()
            return 0

        lax.fori_loop(0, 40, step, 0)
        plsc.subcore_barrier()
        sl = pl.ds(s * 640, 640)
        pltpu.sync_copy(acc.at[sl], out.at[kind, half, sl])
        plsc.subcore_barrier()

    @pl.when(c == 0)
    def _():
        run_pass(0, 0, ewc_hbm, yc0_hbm)
        run_pass(0, 1, ewc_hbm, yc1_hbm)

    @pl.when(c == 1)
    def _():
        run_pass(1, 0, ewo_hbm, yo0_hbm)
        run_pass(1, 1, ewo_hbm, yo1_hbm)


@functools.partial(
    pl.kernel,
    out_type=jax.ShapeDtypeStruct((2, GP, H), f32),
    mesh=_MESH,
    compiler_params=pltpu.CompilerParams(use_tc_tiling_on_sc=False, needs_layout_passes=False),
    scratch_types=[
        pltpu.VMEM((1, 128), i32),
        pltpu.VMEM((128, H), f32),
        pltpu.VMEM_SHARED((GP, H), f32),
    ],
)
def _sc_pool(xc_hbm, xo_hbm, batchb, zeros, out, bidx_v, data_v, acc):
    c = lax.axis_index("c")
    s = lax.axis_index("s")
    _zero_acc(zeros.at[pl.ds(0, 9)], acc, s, 9)
    plsc.subcore_barrier()

    def make_step(x_hbm):
        def step(t, _):
            row = s * 5 + t
            pltpu.sync_copy(batchb.at[pl.ds(row, 1)], bidx_v)
            pltpu.sync_copy(x_hbm.at[pl.ds(row * 128, 128)], data_v)
            pltpu.sync_copy(data_v, acc.at[bidx_v.at[0]], add=True)
            return 0

        return step

    @pl.when(c == 0)
    def _():
        lax.fori_loop(0, 5, make_step(xc_hbm), 0)

    @pl.when(c == 1)
    def _():
        lax.fori_loop(0, 5, make_step(xo_hbm), 0)

    plsc.subcore_barrier()
    _writeback(acc, out, c, s, 9)


def _dot(a, b):
    return jnp.dot(a, b, preferred_element_type=f32)


def _tc_call(body, out_shape):
    return pl.pallas_call(body, out_shape=out_shape)


def _write_y_halves(y_ref, y):
    zpad = jnp.zeros((NP - N, 64), f32)
    y_ref[0, pl.ds(0, N), :] = y[:, :64]
    y_ref[0, pl.ds(N, NP - N), :] = zpad
    y_ref[1, pl.ds(0, N), :] = y[:, 64:]
    y_ref[1, pl.ds(N, NP - N), :] = zpad


def _read_halves(ref, idx=None):
    if idx is None:
        lo = ref[0, pl.ds(0, N), :]
        hi = ref[1, pl.ds(0, N), :]
    else:
        lo = ref[idx, 0, pl.ds(0, N), :]
        hi = ref[idx, 1, pl.ds(0, N), :]
    return jnp.concatenate([lo, hi], axis=1)


def _k0_body(x_ref, degp_ref, wf_ref, bf_ref, w0_ref, y_ref, dinv_ref):
    deg = 1.0 + degp_ref[0, pl.ds(0, N), pl.ds(0, 1)] \
              + degp_ref[1, pl.ds(0, N), pl.ds(0, 1)]
    dinv = lax.rsqrt(deg)
    dinv_ref[...] = dinv
    x0 = jax.nn.relu(_dot(_bn(x_ref[...]), wf_ref[...]) + bf_ref[...])
    y1 = dinv * _dot(_bn(x0), w0_ref[...])
    _write_y_halves(y_ref, y1)


def _kmid_body(z_ref, y_ref, dinv_ref, b_ref, wn_ref, yn_ref):
    dinv = dinv_ref[...]
    x = jax.nn.relu(dinv * (_read_halves(z_ref) + _read_halves(y_ref))
                    + b_ref[...])
    yn = dinv * _dot(_bn(x), wn_ref[...])
    _write_y_halves(yn_ref, yn)


def _k3a_body(z_ref, y_ref, dinv_ref, b_ref, dur_ref, duc_ref, bd_ref,
              wna_ref, bna_ref, x3_ref, na_ref, drc_ref):
    dinv = dinv_ref[...]
    x3 = jax.nn.relu(dinv * (_read_halves(z_ref) + _read_halves(y_ref))
                     + b_ref[...])
    x3_ref[...] = x3
    na_ref[...] = jax.nn.softmax(_dot(x3, wna_ref[...]) + bna_ref[...], axis=-1)
    dr = _dot(x3, dur_ref[...]) + bd_ref[...]
    dc = _dot(x3, duc_ref[...])
    drc_ref[pl.ds(0, N), :] = jnp.concatenate([dr, dc], axis=1)
    drc_ref[pl.ds(N, NP - N), :] = jnp.zeros((NP - N, 2), f32)


def _k3b_body(x3_ref, na_ref, wc_ref, wo_ref, hc_ref, ho_ref):
    x3 = x3_ref[...]
    xc = na_ref[:, pl.ds(0, 1)] * x3
    xo = na_ref[:, pl.ds(1, 1)] * x3
    hc_ref[...] = _dot(_bn(xc), wc_ref[...])
    ho_ref[...] = _dot(_bn(xo), wo_ref[...])


def _k4a_body(degw_ref, dinvc_ref, dinvo_ref):
    degc = 1.0 + degw_ref[0, 0, pl.ds(0, N), pl.ds(0, 1)] \
               + degw_ref[1, 0, pl.ds(0, N), pl.ds(0, 1)]
    dego = 1.0 + degw_ref[0, 1, pl.ds(0, N), pl.ds(0, 1)] \
               + degw_ref[1, 1, pl.ds(0, N), pl.ds(0, 1)]
    dinvc_ref[...] = lax.rsqrt(degc)
    dinvo_ref[...] = lax.rsqrt(dego)


def _k4b_body(h_ref, dinv_ref, y2_ref):
    y = dinv_ref[...] * h_ref[...]
    _write_y_halves(y2_ref, y)


def _k5_body(zw_ref, y2_ref, dinv_ref, b_ref, xk_ref):
    zpad = jnp.zeros((NP - N, H), f32)
    xk = jax.nn.relu(dinv_ref[...] * (_read_halves(zw_ref) + _read_halves(y2_ref))
                     + b_ref[...])
    xk_ref[pl.ds(0, N), :] = xk
    xk_ref[pl.ds(N, NP - N), :] = zpad


def _readout(h, w1, b1, w2, b2):
    h = jax.nn.relu(_dot(_bn(h), w1) + b1)
    h = _dot(_bn(h), w2) + b2
    return jax.nn.log_softmax(h, axis=-1)


def _k6_body(pooled_ref, wc1_ref, bc1_ref, wc2_ref, bc2_ref,
             wo1_ref, bo1_ref, wo2_ref, bo2_ref,
             wco1_ref, bco1_ref, wco2_ref, bco2_ref,
             outc_ref, outo_ref, outco_ref):
    xc = pooled_ref[0, pl.ds(0, G), :]
    xo = pooled_ref[1, pl.ds(0, G), :]
    outc_ref[...] = _readout(xc, wc1_ref[...], bc1_ref[...],
                             wc2_ref[...], bc2_ref[...])
    outo_ref[...] = _readout(xo, wo1_ref[...], bo1_ref[...],
                             wo2_ref[...], bo2_ref[...])
    outco_ref[...] = _readout(xc + xo, wco1_ref[...], bco1_ref[...],
                              wco2_ref[...], bco2_ref[...])


def kernel(x, params, edge_index, batch):
    row = edge_index[0]
    col = edge_index[1]
    padE = jnp.full((EBP * 128 - E,), PAD, i32)
    rowb = jnp.concatenate([row, padE]).reshape(EBP, 128)
    colb = jnp.concatenate([col, padE]).reshape(EBP, 128)
    batchb = jnp.concatenate(
        [batch.astype(i32), jnp.full((NP - N,), G, i32)]).reshape(80, 128)

    zeros = jnp.zeros((640, H), f32)
    zeros64 = jnp.zeros((640, 64), f32)
    zeros16 = jnp.zeros((640, 16), f32)
    ones16 = jnp.ones((128, 16), f32)

    p = params
    row1 = lambda v: v.reshape(1, -1)

    degp = _sc_degree(rowb, ones16, zeros16)

    yhalf = jax.ShapeDtypeStruct((2, NP, 64), f32)

    y1, dinv = _tc_call(
        _k0_body,
        (yhalf, jax.ShapeDtypeStruct((N, 1), f32)),
    )(x, degp, p["W_feat"], row1(p["b_feat"]), p["W_convs"][0])

    z1 = _sc_conv(rowb, colb, y1[0], y1[1], zeros64)
    y2 = _tc_call(_kmid_body, yhalf)(
        z1, y1, dinv, row1(p["b_convs"][0]), p["W_convs"][1])
    z2 = _sc_conv(rowb, colb, y2[0], y2[1], zeros64)
    y3 = _tc_call(_kmid_body, yhalf)(
        z2, y2, dinv, row1(p["b_convs"][1]), p["W_convs"][2])
    z3 = _sc_conv(rowb, colb, y3[0], y3[1], zeros64)

    we = p["W_edge_att"]
    du = we[:, 0] - we[:, 1]
    dur = du[:H].reshape(H, 1)
    duc = du[H:].reshape(H, 1)
    bd = (p["b_edge_att"][0] - p["b_edge_att"][1]).reshape(1, 1)

    x3, na, drc = _tc_call(
        _k3a_body,
        (jax.ShapeDtypeStruct((N, H), f32),
         jax.ShapeDtypeStruct((N, 2), f32),
         jax.ShapeDtypeStruct((NP, 2), f32)),
    )(z3, y3, dinv, row1(p["b_convs"][2]), dur, duc, bd,
      p["W_node_att"], row1(p["b_node_att"]))

    ewc, ewo, degw = _sc_att(drc.reshape(2 * NP), rowb, colb, zeros16)

    hc, ho = _tc_call(
        _k3b_body,
        (jax.ShapeDtypeStruct((N, H), f32), jax.ShapeDtypeStruct((N, H), f32)),
    )(x3, na, p["W_ctx"], p["W_obj"])

    dinvc, dinvo = _tc_call(
        _k4a_body,
        (jax.ShapeDtypeStruct((N, 1), f32), jax.ShapeDtypeStruct((N, 1), f32)),
    )(degw)
    yc2 = _tc_call(_k4b_body, yhalf)(hc, dinvc)
    yo2 = _tc_call(_k4b_body, yhalf)(ho, dinvo)

    zw = _sc_conv_w(rowb, colb, ewc, ewo,
                    yc2[0], yc2[1], yo2[0], yo2[1], zeros64)

    npH = jax.ShapeDtypeStruct((NP, H), f32)
    xc = _tc_call(_k5_body, npH)(zw[0], yc2, dinvc, row1(p["b_ctx"]))
    xo = _tc_call(_k5_body, npH)(zw[1], yo2, dinvo, row1(p["b_obj"]))

    pooled = _sc_pool(xc, xo, batchb, zeros)

    outc, outo, outco = _tc_call(
        _k6_body,
        (jax.ShapeDtypeStruct((G, C), f32),
         jax.ShapeDtypeStruct((G, C), f32),
         jax.ShapeDtypeStruct((G, C), f32)),
    )(pooled,
      p["W_fc1_c"], row1(p["b_fc1_c"]), p["W_fc2_c"], row1(p["b_fc2_c"]),
      p["W_fc1_o"], row1(p["b_fc1_o"]), p["W_fc2_o"], row1(p["b_fc2_o"]),
      p["W_fc1_co"], row1(p["b_fc1_co"]), p["W_fc2_co"], row1(p["b_fc2_co"]))

    return outc, outo, outco
